# Initial kernel scaffold; baseline (speedup 1.0000x reference)
import jax
import jax.numpy as jnp
from jax.experimental import pallas as pl


def kernel(node_reps, edge_reps, graph_rep, subgraph_rep, W1, b1, edge_index, selection):
    raise NotImplementedError("write your pallas kernel here")


# speedup vs baseline: 4.1474x; 4.1474x over previous
import functools

import jax
import jax.numpy as jnp
from jax import lax
from jax.experimental import pallas as pl
from jax.experimental.pallas import tpu as pltpu
from jax.experimental.pallas import tpu_sc as plsc

NEG_INF = -10000000000.0
INV_TEMP = 2.0

_N = 100000
_E = 1600000
_D = 32

_CH = 512
_SUB = 128

_XE = _E // 4
_XB = 3200
_NB = _XE // _XB


def _proj_body(nodes_ref, w_ref, p_ref, q_ref):
    nb = nodes_ref[...]
    w = w_ref[...]
    p_ref[...] = jnp.dot(nb, w[0:_D, :], preferred_element_type=jnp.float32)
    q_ref[...] = jnp.dot(nb, w[_D:2 * _D, :], preferred_element_type=jnp.float32)


def _node_proj(node_reps, W1):
    blk = 5000
    grid = _N // blk
    return pl.pallas_call(
        _proj_body,
        grid=(grid,),
        in_specs=[
            pl.BlockSpec((blk, _D), lambda i: (i, 0)),
            pl.BlockSpec((3 * _D, _D), lambda i: (0, 0)),
        ],
        out_specs=[
            pl.BlockSpec((blk, _D), lambda i: (i, 0)),
            pl.BlockSpec((blk, _D), lambda i: (i, 0)),
        ],
        out_shape=[
            jax.ShapeDtypeStruct((_N, _D), jnp.float32),
            jax.ShapeDtypeStruct((_N, _D), jnp.float32),
        ],
    )(node_reps, W1)


def _sc_gather(P, Q, srcp, dstp, e_pad):
    info = plsc.get_sparse_core_info()
    nc, ns = info.num_cores, info.num_subcores
    nw = nc * ns
    per_w = e_pad // nw
    n_chunks = per_w // _CH
    mesh = plsc.VectorSubcoreMesh(core_axis_name="c", subcore_axis_name="s")

    @functools.partial(
        pl.kernel,
        mesh=mesh,
        compiler_params=pltpu.CompilerParams(use_tc_tiling_on_sc=False),
        out_type=[
            jax.ShapeDtypeStruct((e_pad, _D), jnp.float32),
            jax.ShapeDtypeStruct((e_pad, _D), jnp.float32),
        ],
        scratch_types=[
            pltpu.VMEM((2 * _CH,), jnp.int32),
            pltpu.VMEM((2 * _CH,), jnp.int32),
            pltpu.VMEM((2 * _CH, _D), jnp.float32),
            pltpu.VMEM((2 * _CH, _D), jnp.float32),
            pltpu.SemaphoreType.DMA,
            pltpu.SemaphoreType.DMA,
            pltpu.SemaphoreType.DMA,
        ],
    )
    def k(p_hbm, q_hbm, src_hbm, dst_hbm, sa_hbm, sb_hbm,
          idx_s, idx_d, buf_a, buf_b, sem_g, sem_st, sem_i):
        wid = lax.axis_index("s") * nc + lax.axis_index("c")
        w0 = wid * per_w

        def fire_idx(j, off):
            base = w0 + j * _CH
            pltpu.async_copy(src_hbm.at[pl.ds(base, _CH)],
                             idx_s.at[pl.ds(off, _CH)], sem_i)
            pltpu.async_copy(dst_hbm.at[pl.ds(base, _CH)],
                             idx_d.at[pl.ds(off, _CH)], sem_i)

        def wait_idx():
            pltpu.make_async_copy(src_hbm.at[pl.ds(0, _CH)],
                                  idx_s.at[pl.ds(0, _CH)], sem_i).wait()
            pltpu.make_async_copy(dst_hbm.at[pl.ds(0, _CH)],
                                  idx_d.at[pl.ds(0, _CH)], sem_i).wait()

        def fire_gathers(off):
            for k0 in range(0, _CH, _SUB):
                pltpu.async_copy(p_hbm.at[idx_s.at[pl.ds(off + k0, _SUB)]],
                                 buf_a.at[pl.ds(off + k0, _SUB)], sem_g)
                pltpu.async_copy(q_hbm.at[idx_d.at[pl.ds(off + k0, _SUB)]],
                                 buf_b.at[pl.ds(off + k0, _SUB)], sem_g)

        def drain_gathers():
            for k0 in range(0, _CH, _SUB):
                pltpu.make_async_copy(p_hbm.at[pl.ds(0, _SUB)],
                                      buf_a.at[pl.ds(k0, _SUB)], sem_g).wait()
                pltpu.make_async_copy(q_hbm.at[pl.ds(0, _SUB)],
                                      buf_b.at[pl.ds(k0, _SUB)], sem_g).wait()

        def store(j, off):
            base = w0 + j * _CH
            pltpu.async_copy(buf_a.at[pl.ds(off, _CH)],
                             sa_hbm.at[pl.ds(base, _CH)], sem_st)
            pltpu.async_copy(buf_b.at[pl.ds(off, _CH)],
                             sb_hbm.at[pl.ds(base, _CH)], sem_st)

        def drain_store_pair():
            pltpu.make_async_copy(buf_a.at[pl.ds(0, _CH)],
                                  sa_hbm.at[pl.ds(w0, _CH)], sem_st).wait()
            pltpu.make_async_copy(buf_b.at[pl.ds(0, _CH)],
                                  sb_hbm.at[pl.ds(w0, _CH)], sem_st).wait()

        fire_idx(0, 0)
        wait_idx()
        fire_gathers(0)
        fire_idx(1, _CH)

        def body(j, c):
            p_off = (j % 2) * _CH
            q_off = _CH - p_off
            drain_gathers()

            @pl.when(j >= 1)
            def _():
                drain_store_pair()

            @pl.when(j + 1 < n_chunks)
            def _():
                wait_idx()
                fire_gathers(q_off)

            @pl.when(j + 2 < n_chunks)
            def _():
                fire_idx(j + 2, p_off)

            store(j, p_off)
            return c

        lax.fori_loop(0, n_chunks, body, 0)
        drain_store_pair()

    return k(P, Q, srcp, dstp)


def _score_body(sa_ref, sb_ref, er_ref, w_ref, b_ref, gr_ref, sr_ref, sel_ref,
                s_ref, gmax_ref, acc_ref):
    i = pl.program_id(0)
    w = w_ref[...]
    c = w[2 * _D:3 * _D, :]
    rows = lax.broadcasted_iota(jnp.int32, (128, 128), 0) // _D
    cols = lax.broadcasted_iota(jnp.int32, (128, 128), 1) // _D
    wbd = jnp.where(rows == cols, jnp.tile(c, (4, 4)), 0.0)
    btile = jnp.tile(b_ref[...], (1, 4))
    g = gr_ref[...] - sr_ref[...]
    gtile = jnp.tile(g, (1, 4))
    h = (sa_ref[...] + sb_ref[...]
         + jnp.dot(er_ref[...], wbd, preferred_element_type=jnp.float32)
         + btile)
    e = jnp.where(h > 0.0, h, jnp.exp(h) - 1.0)
    eb = e.astype(jnp.bfloat16).astype(jnp.float32)
    gb = gtile.astype(jnp.bfloat16).astype(jnp.float32)
    mt = jnp.transpose(eb * gb)
    m0 = -jnp.inf
    for j in range(4):
        s_j = jnp.sum(mt[_D * j:_D * (j + 1), :], axis=0)
        s_j = jnp.where(sel_ref[j, :] > 0, NEG_INF, s_j)
        s_ref[j, :] = s_j
        m0 = jnp.maximum(m0, jnp.max(s_j))
    prev = jnp.where(i == 0, -jnp.inf, acc_ref[0, 0])
    acc_ref[0, 0] = jnp.maximum(prev, m0)

    @pl.when(i == pl.num_programs(0) - 1)
    def _():
        gmax_ref[...] = jnp.full((1, 128), acc_ref[0, 0], jnp.float32)


def _scores(SAp, SBp, edge_p, W1, b1, graph_rep, subgraph_rep, sel4):
    return pl.pallas_call(
        _score_body,
        grid=(_NB,),
        in_specs=[
            pl.BlockSpec((_XB, 128), lambda i: (i, 0)),
            pl.BlockSpec((_XB, 128), lambda i: (i, 0)),
            pl.BlockSpec((_XB, 128), lambda i: (i, 0)),
            pl.BlockSpec((3 * _D, _D), lambda i: (0, 0)),
            pl.BlockSpec((1, _D), lambda i: (0, 0)),
            pl.BlockSpec((1, _D), lambda i: (0, 0)),
            pl.BlockSpec((1, _D), lambda i: (0, 0)),
            pl.BlockSpec((4, _XB), lambda i: (0, i)),
        ],
        out_specs=[
            pl.BlockSpec((4, _XB), lambda i: (0, i)),
            pl.BlockSpec((1, 128), lambda i: (0, 0)),
        ],
        out_shape=[
            jax.ShapeDtypeStruct((4, _XE), jnp.float32),
            jax.ShapeDtypeStruct((1, 128), jnp.float32),
        ],
        scratch_shapes=[pltpu.SMEM((1, 1), jnp.float32)],
    )(SAp, SBp, edge_p, W1,
      b1.reshape(1, _D), graph_rep.reshape(1, _D), subgraph_rep.reshape(1, _D),
      sel4)


def _exp_body(s_ref, gm_ref, p_ref, tot_ref, acc_ref):
    i = pl.program_id(0)
    m = gm_ref[0, 0]
    p = jnp.exp((s_ref[...] - m) * INV_TEMP)
    p_ref[...] = p
    prev = jnp.where(i == 0, 0.0, acc_ref[0, 0])
    acc_ref[0, 0] = prev + jnp.sum(p)

    @pl.when(i == pl.num_programs(0) - 1)
    def _():
        tot_ref[...] = jnp.full((1, 128), acc_ref[0, 0], jnp.float32)


def _exp_pass(scores, gmax):
    return pl.pallas_call(
        _exp_body,
        grid=(_NB,),
        in_specs=[
            pl.BlockSpec((4, _XB), lambda i: (0, i)),
            pl.BlockSpec((1, 128), lambda i: (0, 0)),
        ],
        out_specs=[
            pl.BlockSpec((4, _XB), lambda i: (0, i)),
            pl.BlockSpec((1, 128), lambda i: (0, 0)),
        ],
        out_shape=[
            jax.ShapeDtypeStruct((4, _XE), jnp.float32),
            jax.ShapeDtypeStruct((1, 128), jnp.float32),
        ],
        scratch_shapes=[pltpu.SMEM((1, 1), jnp.float32)],
    )(scores, gmax)


def _norm_body(p_ref, tot_ref, o_ref):
    o_ref[...] = p_ref[...] * (1.0 / tot_ref[0, 0])


def _normalize(p, tot):
    return pl.pallas_call(
        _norm_body,
        grid=(_NB,),
        in_specs=[
            pl.BlockSpec((4, _XB), lambda i: (0, i)),
            pl.BlockSpec((1, 128), lambda i: (0, 0)),
        ],
        out_specs=pl.BlockSpec((4, _XB), lambda i: (0, i)),
        out_shape=jax.ShapeDtypeStruct((4, _XE), jnp.float32),
    )(p, tot)


def kernel(node_reps, edge_reps, graph_rep, subgraph_rep, W1, b1, edge_index,
           selection):
    info = plsc.get_sparse_core_info()
    nw = info.num_cores * info.num_subcores
    gran = nw * _CH
    e_pad = ((_E + gran - 1) // gran) * gran

    P, Q = _node_proj(node_reps, W1)
    srcp = jnp.pad(edge_index[0], (0, e_pad - _E))
    dstp = jnp.pad(edge_index[1], (0, e_pad - _E))
    SA, SB = _sc_gather(P, Q, srcp, dstp, e_pad)

    SAp = SA.reshape(e_pad // 4, 128)
    SBp = SB.reshape(e_pad // 4, 128)
    edge_p = edge_reps.reshape(_XE, 128)
    sel4 = jnp.transpose(selection.astype(jnp.int32).reshape(_XE, 4))

    scores, gmax = _scores(SAp, SBp, edge_p, W1, b1, graph_rep, subgraph_rep,
                           sel4)
    p, tot = _exp_pass(scores, gmax)
    out = _normalize(p, tot)
    return jnp.transpose(out).reshape(_E)


# speedup vs baseline: 4.2219x; 1.0180x over previous
import functools

import jax
import jax.numpy as jnp
from jax import lax
from jax.experimental import pallas as pl
from jax.experimental.pallas import tpu as pltpu
from jax.experimental.pallas import tpu_sc as plsc

NEG_INF = -10000000000.0
INV_TEMP = 2.0

_N = 100000
_E = 1600000
_D = 32

_CH = 512
_SUB = 128

_XE = _E // 4
_XB = 3200
_NB = _XE // _XB


def _proj_body(nodes_ref, w_ref, p_ref, q_ref):
    nb = nodes_ref[...]
    w = w_ref[...]
    p_ref[...] = jnp.dot(nb, w[0:_D, :], preferred_element_type=jnp.float32)
    q_ref[...] = jnp.dot(nb, w[_D:2 * _D, :], preferred_element_type=jnp.float32)


def _node_proj(node_reps, W1):
    blk = 5000
    grid = _N // blk
    return pl.pallas_call(
        _proj_body,
        grid=(grid,),
        in_specs=[
            pl.BlockSpec((blk, _D), lambda i: (i, 0)),
            pl.BlockSpec((3 * _D, _D), lambda i: (0, 0)),
        ],
        out_specs=[
            pl.BlockSpec((blk, _D), lambda i: (i, 0)),
            pl.BlockSpec((blk, _D), lambda i: (i, 0)),
        ],
        out_shape=[
            jax.ShapeDtypeStruct((_N, _D), jnp.float32),
            jax.ShapeDtypeStruct((_N, _D), jnp.float32),
        ],
    )(node_reps, W1)


def _sc_gather(P, Q, srcp, dstp, e_pad):
    info = plsc.get_sparse_core_info()
    nc, ns = info.num_cores, info.num_subcores
    nw = nc * ns
    per_w = e_pad // nw
    n_chunks = per_w // _CH
    mesh = plsc.VectorSubcoreMesh(core_axis_name="c", subcore_axis_name="s")

    @functools.partial(
        pl.kernel,
        mesh=mesh,
        compiler_params=pltpu.CompilerParams(use_tc_tiling_on_sc=False),
        out_type=[
            jax.ShapeDtypeStruct((e_pad, _D), jnp.float32),
            jax.ShapeDtypeStruct((e_pad, _D), jnp.float32),
        ],
        scratch_types=[
            pltpu.VMEM((_CH,), jnp.int32),
            pltpu.VMEM((_CH,), jnp.int32),
            pltpu.VMEM((_CH, _D), jnp.float32),
            pltpu.VMEM((_CH, _D), jnp.float32),
            pltpu.SemaphoreType.DMA,
            pltpu.SemaphoreType.DMA,
        ],
    )
    def k(p_hbm, q_hbm, src_hbm, dst_hbm, sa_hbm, sb_hbm,
          idx_s, idx_d, buf_a, buf_b, sem_a, sem_b):
        wid = lax.axis_index("s") * nc + lax.axis_index("c")

        def chunk(j, carry):
            base = wid * per_w + j * _CH
            pltpu.sync_copy(src_hbm.at[pl.ds(base, _CH)], idx_s)
            pltpu.sync_copy(dst_hbm.at[pl.ds(base, _CH)], idx_d)
            handles = []
            for k0 in range(0, _CH, _SUB):
                handles.append(pltpu.async_copy(
                    p_hbm.at[idx_s.at[pl.ds(k0, _SUB)]],
                    buf_a.at[pl.ds(k0, _SUB)], sem_a))
                handles.append(pltpu.async_copy(
                    q_hbm.at[idx_d.at[pl.ds(k0, _SUB)]],
                    buf_b.at[pl.ds(k0, _SUB)], sem_b))
            for h in handles:
                h.wait()
            pltpu.sync_copy(buf_a, sa_hbm.at[pl.ds(base, _CH)])
            pltpu.sync_copy(buf_b, sb_hbm.at[pl.ds(base, _CH)])
            return carry

        lax.fori_loop(0, n_chunks, chunk, 0)

    return k(P, Q, srcp, dstp)


def _score_body(sa_ref, sb_ref, er_ref, w_ref, b_ref, gr_ref, sr_ref, sel_ref,
                s_ref, gmax_ref, acc_ref):
    i = pl.program_id(0)
    w = w_ref[...]
    c = w[2 * _D:3 * _D, :]
    rows = lax.broadcasted_iota(jnp.int32, (128, 128), 0) // _D
    cols = lax.broadcasted_iota(jnp.int32, (128, 128), 1) // _D
    wbd = jnp.where(rows == cols, jnp.tile(c, (4, 4)), 0.0)
    btile = jnp.tile(b_ref[...], (1, 4))
    g = gr_ref[...] - sr_ref[...]
    gtile = jnp.tile(g, (1, 4))
    h = (sa_ref[...] + sb_ref[...]
         + jnp.dot(er_ref[...], wbd, preferred_element_type=jnp.float32)
         + btile)
    e = jnp.where(h > 0.0, h, jnp.exp(h) - 1.0)
    eb = e.astype(jnp.bfloat16).astype(jnp.float32)
    gb = gtile.astype(jnp.bfloat16).astype(jnp.float32)
    mt = jnp.transpose(eb * gb)
    m0 = -jnp.inf
    for j in range(4):
        s_j = jnp.sum(mt[_D * j:_D * (j + 1), :], axis=0)
        s_j = jnp.where(sel_ref[j, :] > 0, NEG_INF, s_j)
        s_ref[j, :] = s_j
        m0 = jnp.maximum(m0, jnp.max(s_j))
    prev = jnp.where(i == 0, -jnp.inf, acc_ref[0, 0])
    acc_ref[0, 0] = jnp.maximum(prev, m0)

    @pl.when(i == pl.num_programs(0) - 1)
    def _():
        gmax_ref[...] = jnp.full((1, 128), acc_ref[0, 0], jnp.float32)


def _scores(SAp, SBp, edge_p, W1, b1, graph_rep, subgraph_rep, sel4):
    return pl.pallas_call(
        _score_body,
        grid=(_NB,),
        in_specs=[
            pl.BlockSpec((_XB, 128), lambda i: (i, 0)),
            pl.BlockSpec((_XB, 128), lambda i: (i, 0)),
            pl.BlockSpec((_XB, 128), lambda i: (i, 0)),
            pl.BlockSpec((3 * _D, _D), lambda i: (0, 0)),
            pl.BlockSpec((1, _D), lambda i: (0, 0)),
            pl.BlockSpec((1, _D), lambda i: (0, 0)),
            pl.BlockSpec((1, _D), lambda i: (0, 0)),
            pl.BlockSpec((4, _XB), lambda i: (0, i)),
        ],
        out_specs=[
            pl.BlockSpec((4, _XB), lambda i: (0, i)),
            pl.BlockSpec((1, 128), lambda i: (0, 0)),
        ],
        out_shape=[
            jax.ShapeDtypeStruct((4, _XE), jnp.float32),
            jax.ShapeDtypeStruct((1, 128), jnp.float32),
        ],
        scratch_shapes=[pltpu.SMEM((1, 1), jnp.float32)],
    )(SAp, SBp, edge_p, W1,
      b1.reshape(1, _D), graph_rep.reshape(1, _D), subgraph_rep.reshape(1, _D),
      sel4)


def _exp_body(s_ref, gm_ref, p_ref, tot_ref, acc_ref):
    i = pl.program_id(0)
    m = gm_ref[0, 0]
    p = jnp.exp((s_ref[...] - m) * INV_TEMP)
    p_ref[...] = p
    prev = jnp.where(i == 0, 0.0, acc_ref[0, 0])
    acc_ref[0, 0] = prev + jnp.sum(p)

    @pl.when(i == pl.num_programs(0) - 1)
    def _():
        tot_ref[...] = jnp.full((1, 128), acc_ref[0, 0], jnp.float32)


def _exp_pass(scores, gmax):
    return pl.pallas_call(
        _exp_body,
        grid=(_NB,),
        in_specs=[
            pl.BlockSpec((4, _XB), lambda i: (0, i)),
            pl.BlockSpec((1, 128), lambda i: (0, 0)),
        ],
        out_specs=[
            pl.BlockSpec((4, _XB), lambda i: (0, i)),
            pl.BlockSpec((1, 128), lambda i: (0, 0)),
        ],
        out_shape=[
            jax.ShapeDtypeStruct((4, _XE), jnp.float32),
            jax.ShapeDtypeStruct((1, 128), jnp.float32),
        ],
        scratch_shapes=[pltpu.SMEM((1, 1), jnp.float32)],
    )(scores, gmax)


def _norm_body(p_ref, tot_ref, o_ref):
    o_ref[...] = p_ref[...] * (1.0 / tot_ref[0, 0])


def _normalize(p, tot):
    return pl.pallas_call(
        _norm_body,
        grid=(_NB,),
        in_specs=[
            pl.BlockSpec((4, _XB), lambda i: (0, i)),
            pl.BlockSpec((1, 128), lambda i: (0, 0)),
        ],
        out_specs=pl.BlockSpec((4, _XB), lambda i: (0, i)),
        out_shape=jax.ShapeDtypeStruct((4, _XE), jnp.float32),
    )(p, tot)


def kernel(node_reps, edge_reps, graph_rep, subgraph_rep, W1, b1, edge_index,
           selection):
    info = plsc.get_sparse_core_info()
    nw = info.num_cores * info.num_subcores
    gran = nw * _CH
    e_pad = ((_E + gran - 1) // gran) * gran

    P, Q = _node_proj(node_reps, W1)
    srcp = jnp.pad(edge_index[0], (0, e_pad - _E))
    dstp = jnp.pad(edge_index[1], (0, e_pad - _E))
    SA, SB = _sc_gather(P, Q, srcp, dstp, e_pad)

    SAp = SA.reshape(e_pad // 4, 128)
    SBp = SB.reshape(e_pad // 4, 128)
    edge_p = edge_reps.reshape(_XE, 128)
    sel4 = jnp.transpose(selection.astype(jnp.int32).reshape(_XE, 4))

    scores, gmax = _scores(SAp, SBp, edge_p, W1, b1, graph_rep, subgraph_rep,
                           sel4)
    p, tot = _exp_pass(scores, gmax)
    out = _normalize(p, tot)
    return jnp.transpose(out).reshape(_E)
